# Initial kernel scaffold; baseline (speedup 1.0000x reference)
import jax
import jax.numpy as jnp
from jax.experimental import pallas as pl


def kernel(x, edge_index, edge_weight, W1, b1, W2, b2, fc1_W, fc1_b, fc2_W, fc2_b):
    raise NotImplementedError("write your pallas kernel here")


# speedup vs baseline: 4.6895x; 4.6895x over previous
import dataclasses
import functools

import jax
import jax.numpy as jnp
from jax import lax
from jax.experimental import pallas as pl
from jax.experimental.pallas import tpu as pltpu
from jax.experimental.pallas import tpu_sc as plsc

_NC = 2
_NS = 16
_L = 16
_NW = _NC * _NS
_CHUNK = 128


def _matmul_tc(x, W):
    n, f = x.shape
    h = W.shape[1]
    br = 1000
    grid = n // br

    def body(x_ref, w_ref, o_ref):
        o_ref[...] = lax.dot_general(
            x_ref[...], w_ref[...], (((1,), (0,)), ((), ())),
            preferred_element_type=jnp.float32,
            precision=lax.Precision.HIGHEST)

    return pl.pallas_call(
        body,
        grid=(grid,),
        in_specs=[pl.BlockSpec((br, f), lambda i: (i, 0)),
                  pl.BlockSpec((f, h), lambda i: (0, 0))],
        out_specs=pl.BlockSpec((br, h), lambda i: (i, 0)),
        out_shape=jax.ShapeDtypeStruct((n, h), jnp.float32),
    )(x, W)


def _edge_pass_sc(h, src, dst, w, n_nodes):
    ep = src.shape[0]
    per_w = ep // _NW
    n_chunks = per_w // _CHUNK
    rows_per_tile = -(-n_nodes // (_NS * 8)) * 8
    n_pad = rows_per_tile * _NS
    hdim = h.shape[1]

    mesh = plsc.VectorSubcoreMesh(core_axis_name="c", subcore_axis_name="s",
                                  num_cores=_NC, num_subcores=_NS)
    cp = pltpu.CompilerParams()
    if "needs_layout_passes" in pltpu.CompilerParams.__dataclass_fields__:
        cp = dataclasses.replace(cp, needs_layout_passes=False)
    if "use_tc_tiling_on_sc" in pltpu.CompilerParams.__dataclass_fields__:
        cp = dataclasses.replace(cp, use_tc_tiling_on_sc=False)

    @functools.partial(
        pl.kernel,
        compiler_params=cp,
        out_type=jax.ShapeDtypeStruct((_NC, n_pad, hdim), jnp.float32),
        mesh=mesh,
        scratch_types=[
            pltpu.VMEM((_CHUNK,), jnp.int32),
            pltpu.VMEM((_CHUNK,), jnp.int32),
            pltpu.VMEM((_CHUNK,), jnp.float32),
            pltpu.VMEM((_CHUNK, hdim), jnp.float32),
            pltpu.VMEM((rows_per_tile, hdim), jnp.float32),
            pltpu.VMEM_SHARED((n_pad, hdim), jnp.float32),
            pltpu.SemaphoreType.DMA,
        ],
    )
    def ker(h_hbm, src_hbm, dst_hbm, w_hbm, out_hbm,
            src_v, dst_v, w_v, rows_v, zero_v, agg_sh, sem):
        c = lax.axis_index("c")
        s = lax.axis_index("s")
        wid = c * _NS + s

        @pl.loop(0, rows_per_tile)
        def _(i):
            @pl.loop(0, hdim // _L)
            def _(j):
                zero_v[i, pl.ds(j * _L, _L)] = jnp.zeros((_L,), jnp.float32)

        pltpu.sync_copy(zero_v,
                        agg_sh.at[pl.ds(s * rows_per_tile, rows_per_tile)])
        plsc.subcore_barrier()

        base = wid * per_w

        @pl.loop(0, n_chunks)
        def _(ci):
            off = base + ci * _CHUNK
            pltpu.sync_copy(src_hbm.at[pl.ds(off, _CHUNK)], src_v)
            pltpu.sync_copy(dst_hbm.at[pl.ds(off, _CHUNK)], dst_v)
            pltpu.sync_copy(w_hbm.at[pl.ds(off, _CHUNK)], w_v)
            pltpu.async_copy(h_hbm.at[src_v], rows_v, sem).wait()

            @pl.loop(0, _CHUNK)
            def _(e):
                widx = jnp.full((_L,), e, jnp.int32)
                wvec = plsc.load_gather(w_v, [widx])
                @pl.loop(0, hdim // _L)
                def _(j):
                    rows_v[e, pl.ds(j * _L, _L)] = (
                        rows_v[e, pl.ds(j * _L, _L)] * wvec)

            pltpu.sync_copy(rows_v, agg_sh.at[dst_v], add=True)

        plsc.subcore_barrier()
        pltpu.sync_copy(
            agg_sh.at[pl.ds(s * rows_per_tile, rows_per_tile)],
            out_hbm.at[c].at[pl.ds(s * rows_per_tile, rows_per_tile)])

    return ker(h, src, dst, w)


def _elu(t):
    return jnp.where(t > 0, t, jnp.exp(jnp.minimum(t, 0.0)) - 1.0)


def _mid_layer_tc(parts, b1, W2, n):
    hdim = parts.shape[2]
    h2 = W2.shape[1]
    br = 1000
    grid = n // br

    def body(p_ref, b_ref, w_ref, o_ref):
        t = p_ref[0] + p_ref[1] + b_ref[...]
        t = _elu(t)
        o_ref[...] = lax.dot_general(
            t, w_ref[...], (((1,), (0,)), ((), ())),
            preferred_element_type=jnp.float32,
            precision=lax.Precision.HIGHEST)

    return pl.pallas_call(
        body,
        grid=(grid,),
        in_specs=[pl.BlockSpec((2, br, hdim), lambda i: (0, i, 0)),
                  pl.BlockSpec((1, hdim), lambda i: (0, 0)),
                  pl.BlockSpec((hdim, h2), lambda i: (0, 0))],
        out_specs=pl.BlockSpec((br, h2), lambda i: (i, 0)),
        out_shape=jax.ShapeDtypeStruct((n, h2), jnp.float32),
    )(parts, b1.reshape(1, -1), W2)


def _head_tc(parts, b2, fc1_W, fc1_b, fc2_W, fc2_b, n):
    hdim = parts.shape[2]
    fc1 = fc1_W.shape[1]
    out_dim = fc2_W.shape[1]
    br = 1000
    grid = n // br

    def body(p_ref, b_ref, w1_ref, c1_ref, w2_ref, c2_ref, o_ref, acc_ref):
        i = pl.program_id(0)
        t = p_ref[0] + p_ref[1] + b_ref[...]
        t = _elu(t)
        part = jnp.sum(t, axis=0, keepdims=True)

        @pl.when(i == 0)
        def _():
            acc_ref[...] = part

        @pl.when(i > 0)
        def _():
            acc_ref[...] = acc_ref[...] + part

        @pl.when(i == pl.num_programs(0) - 1)
        def _():
            z = lax.dot_general(
                acc_ref[...], w1_ref[...], (((1,), (0,)), ((), ())),
                preferred_element_type=jnp.float32,
                precision=lax.Precision.HIGHEST) + c1_ref[...]
            z = jnp.maximum(z, 0.0)
            y = lax.dot_general(
                z, w2_ref[...], (((1,), (0,)), ((), ())),
                preferred_element_type=jnp.float32,
                precision=lax.Precision.HIGHEST) + c2_ref[...]
            o_ref[...] = 1.0 / (1.0 + jnp.exp(-y))

    return pl.pallas_call(
        body,
        grid=(grid,),
        in_specs=[pl.BlockSpec((2, br, hdim), lambda i: (0, i, 0)),
                  pl.BlockSpec((1, hdim), lambda i: (0, 0)),
                  pl.BlockSpec((hdim, fc1), lambda i: (0, 0)),
                  pl.BlockSpec((1, fc1), lambda i: (0, 0)),
                  pl.BlockSpec((fc1, out_dim), lambda i: (0, 0)),
                  pl.BlockSpec((1, out_dim), lambda i: (0, 0))],
        out_specs=pl.BlockSpec((1, out_dim), lambda i: (0, 0)),
        out_shape=jax.ShapeDtypeStruct((1, out_dim), jnp.float32),
        scratch_shapes=[pltpu.VMEM((1, hdim), jnp.float32)],
    )(parts, b2.reshape(1, -1), fc1_W, fc1_b.reshape(1, -1),
      fc2_W, fc2_b.reshape(1, -1))


def kernel(x, edge_index, edge_weight, W1, b1, W2, b2,
           fc1_W, fc1_b, fc2_W, fc2_b):
    n = x.shape[0]
    e = edge_index.shape[1]
    src = edge_index[0]
    dst = edge_index[1]

    unit = _NW * _CHUNK
    ep = ((e + unit - 1) // unit) * unit
    pad = ep - e
    if pad:
        src = jnp.concatenate([src, jnp.zeros((pad,), jnp.int32)])
        dst = jnp.concatenate([dst, jnp.zeros((pad,), jnp.int32)])
        edge_weight = jnp.concatenate(
            [edge_weight, jnp.zeros((pad,), jnp.float32)])

    h1 = _matmul_tc(x, W1)
    p1 = _edge_pass_sc(h1, src, dst, edge_weight, n)
    h2 = _mid_layer_tc(p1, b1, W2, n)
    p2 = _edge_pass_sc(h2, src, dst, edge_weight, n)
    out = _head_tc(p2, b2, fc1_W, fc1_b, fc2_W, fc2_b, n)
    return out.reshape(-1)


# speedup vs baseline: 6.3945x; 1.3636x over previous
import dataclasses
import functools

import jax
import jax.numpy as jnp
from jax import lax
from jax.experimental import pallas as pl
from jax.experimental.pallas import tpu as pltpu
from jax.experimental.pallas import tpu_sc as plsc

_NC = 2
_NS = 16
_L = 16
_NW = _NC * _NS
_CHUNK = 128


def _matmul_tc(x, W):
    n, f = x.shape
    h = W.shape[1]
    br = 1000
    grid = n // br

    def body(x_ref, w_ref, o_ref):
        o_ref[...] = lax.dot_general(
            x_ref[...], w_ref[...], (((1,), (0,)), ((), ())),
            preferred_element_type=jnp.float32,
            precision=lax.Precision.HIGHEST)

    return pl.pallas_call(
        body,
        grid=(grid,),
        in_specs=[pl.BlockSpec((br, f), lambda i: (i, 0)),
                  pl.BlockSpec((f, h), lambda i: (0, 0))],
        out_specs=pl.BlockSpec((br, h), lambda i: (i, 0)),
        out_shape=jax.ShapeDtypeStruct((n, h), jnp.float32),
    )(x, W)


def _bcast_lane(v, j):
    idx = jnp.full((_L, 1), j, jnp.int32)
    return lax.gather(
        v, idx,
        lax.GatherDimensionNumbers(offset_dims=(), collapsed_slice_dims=(0,),
                                   start_index_map=(0,)),
        (1,), mode=lax.GatherScatterMode.PROMISE_IN_BOUNDS)


def _edge_pass_sc(h, pk, dst, n_nodes, ep):
    per_w = ep // _NW
    n_chunks = per_w // _CHUNK
    rows_per_tile = -(-n_nodes // (_NS * 8)) * 8
    n_pad = rows_per_tile * _NS
    hdim = h.shape[1]

    mesh = plsc.VectorSubcoreMesh(core_axis_name="c", subcore_axis_name="s",
                                  num_cores=_NC, num_subcores=_NS)
    cp = pltpu.CompilerParams()
    if "needs_layout_passes" in pltpu.CompilerParams.__dataclass_fields__:
        cp = dataclasses.replace(cp, needs_layout_passes=False)
    if "use_tc_tiling_on_sc" in pltpu.CompilerParams.__dataclass_fields__:
        cp = dataclasses.replace(cp, use_tc_tiling_on_sc=False)

    @functools.partial(
        pl.kernel,
        compiler_params=cp,
        out_type=jax.ShapeDtypeStruct((_NC, n_pad, hdim), jnp.float32),
        mesh=mesh,
        scratch_types=[
            pltpu.VMEM((3, _CHUNK), jnp.int32),
            pltpu.VMEM((_CHUNK,), jnp.int32),
            pltpu.VMEM((_CHUNK, hdim), jnp.float32),
            pltpu.VMEM_SHARED((n_pad, hdim), jnp.float32),
            pltpu.SemaphoreType.DMA,
        ],
    )
    def ker(h_hbm, pk_hbm, dst_hbm, out_hbm, pk_v, dst_v, rows_v, agg_sh,
            sem):
        c = lax.axis_index("c")
        s = lax.axis_index("s")
        wid = c * _NS + s

        @pl.loop(0, _CHUNK)
        def _(i):
            for j in range(hdim // _L):
                rows_v[i, pl.ds(j * _L, _L)] = jnp.zeros((_L,), jnp.float32)

        @pl.loop(0, rows_per_tile // _CHUNK)
        def _(i):
            pltpu.sync_copy(
                rows_v, agg_sh.at[pl.ds(s * rows_per_tile + i * _CHUNK,
                                        _CHUNK)])
        plsc.subcore_barrier()

        base = wid * n_chunks

        @pl.loop(0, n_chunks)
        def _(ci):
            pltpu.sync_copy(pk_hbm.at[base + ci], pk_v)
            pltpu.sync_copy(
                dst_hbm.at[pl.ds((base + ci) * _CHUNK, _CHUNK)], dst_v)
            pltpu.async_copy(h_hbm.at[pk_v.at[0]], rows_v, sem).wait()

            for g in range(_CHUNK // _L):
                w16 = plsc.bitcast(pk_v[2, pl.ds(g * _L, _L)], jnp.float32)
                for j in range(_L):
                    e = g * _L + j
                    wvec = _bcast_lane(w16, j)
                    for half in range(hdim // _L):
                        rows_v[e, pl.ds(half * _L, _L)] = (
                            rows_v[e, pl.ds(half * _L, _L)] * wvec)

            pltpu.sync_copy(rows_v, agg_sh.at[dst_v], add=True)

        plsc.subcore_barrier()
        pltpu.sync_copy(
            agg_sh.at[pl.ds(s * rows_per_tile, rows_per_tile)],
            out_hbm.at[c].at[pl.ds(s * rows_per_tile, rows_per_tile)])

    return ker(h, pk, dst)


def _elu(t):
    return jnp.where(t > 0, t, jnp.exp(jnp.minimum(t, 0.0)) - 1.0)


def _mid_layer_tc(parts, b1, W2, n):
    hdim = parts.shape[2]
    h2 = W2.shape[1]
    br = 1000
    grid = n // br

    def body(p_ref, b_ref, w_ref, o_ref):
        t = p_ref[0] + p_ref[1] + b_ref[...]
        t = _elu(t)
        o_ref[...] = lax.dot_general(
            t, w_ref[...], (((1,), (0,)), ((), ())),
            preferred_element_type=jnp.float32,
            precision=lax.Precision.HIGHEST)

    return pl.pallas_call(
        body,
        grid=(grid,),
        in_specs=[pl.BlockSpec((2, br, hdim), lambda i: (0, i, 0)),
                  pl.BlockSpec((1, hdim), lambda i: (0, 0)),
                  pl.BlockSpec((hdim, h2), lambda i: (0, 0))],
        out_specs=pl.BlockSpec((br, h2), lambda i: (i, 0)),
        out_shape=jax.ShapeDtypeStruct((n, h2), jnp.float32),
    )(parts, b1.reshape(1, -1), W2)


def _head_tc(parts, b2, fc1_W, fc1_b, fc2_W, fc2_b, n):
    hdim = parts.shape[2]
    fc1 = fc1_W.shape[1]
    out_dim = fc2_W.shape[1]
    br = 1000
    grid = n // br

    def body(p_ref, b_ref, w1_ref, c1_ref, w2_ref, c2_ref, o_ref, acc_ref):
        i = pl.program_id(0)
        t = p_ref[0] + p_ref[1] + b_ref[...]
        t = _elu(t)
        part = jnp.sum(t, axis=0, keepdims=True)

        @pl.when(i == 0)
        def _():
            acc_ref[...] = part

        @pl.when(i > 0)
        def _():
            acc_ref[...] = acc_ref[...] + part

        @pl.when(i == pl.num_programs(0) - 1)
        def _():
            z = lax.dot_general(
                acc_ref[...], w1_ref[...], (((1,), (0,)), ((), ())),
                preferred_element_type=jnp.float32,
                precision=lax.Precision.HIGHEST) + c1_ref[...]
            z = jnp.maximum(z, 0.0)
            y = lax.dot_general(
                z, w2_ref[...], (((1,), (0,)), ((), ())),
                preferred_element_type=jnp.float32,
                precision=lax.Precision.HIGHEST) + c2_ref[...]
            o_ref[...] = 1.0 / (1.0 + jnp.exp(-y))

    return pl.pallas_call(
        body,
        grid=(grid,),
        in_specs=[pl.BlockSpec((2, br, hdim), lambda i: (0, i, 0)),
                  pl.BlockSpec((1, hdim), lambda i: (0, 0)),
                  pl.BlockSpec((hdim, fc1), lambda i: (0, 0)),
                  pl.BlockSpec((1, fc1), lambda i: (0, 0)),
                  pl.BlockSpec((fc1, out_dim), lambda i: (0, 0)),
                  pl.BlockSpec((1, out_dim), lambda i: (0, 0))],
        out_specs=pl.BlockSpec((1, out_dim), lambda i: (0, 0)),
        out_shape=jax.ShapeDtypeStruct((1, out_dim), jnp.float32),
        scratch_shapes=[pltpu.VMEM((1, hdim), jnp.float32)],
    )(parts, b2.reshape(1, -1), fc1_W, fc1_b.reshape(1, -1),
      fc2_W, fc2_b.reshape(1, -1))


def kernel(x, edge_index, edge_weight, W1, b1, W2, b2,
           fc1_W, fc1_b, fc2_W, fc2_b):
    n = x.shape[0]
    e = edge_index.shape[1]
    src = edge_index[0]
    dst = edge_index[1]

    unit = _NW * _CHUNK
    ep = ((e + unit - 1) // unit) * unit
    pad = ep - e
    if pad:
        src = jnp.concatenate([src, jnp.zeros((pad,), jnp.int32)])
        dst = jnp.concatenate([dst, jnp.zeros((pad,), jnp.int32)])
        edge_weight = jnp.concatenate(
            [edge_weight, jnp.zeros((pad,), jnp.float32)])

    pk = jnp.stack(
        [src.reshape(-1, _CHUNK), dst.reshape(-1, _CHUNK),
         lax.bitcast_convert_type(edge_weight, jnp.int32).reshape(-1, _CHUNK)],
        axis=1)

    h1 = _matmul_tc(x, W1)
    p1 = _edge_pass_sc(h1, pk, dst, n, ep)
    h2 = _mid_layer_tc(p1, b1, W2, n)
    p2 = _edge_pass_sc(h2, pk, dst, n, ep)
    out = _head_tc(p2, b2, fc1_W, fc1_b, fc2_W, fc2_b, n)
    return out.reshape(-1)


# speedup vs baseline: 7.1827x; 1.1233x over previous
import dataclasses
import functools

import jax
import jax.numpy as jnp
from jax import lax
from jax.experimental import pallas as pl
from jax.experimental.pallas import tpu as pltpu
from jax.experimental.pallas import tpu_sc as plsc

_NC = 2
_NS = 16
_L = 16
_NW = _NC * _NS
_CHUNK = 128


def _matmul_tc(x, W):
    n, f = x.shape
    h = W.shape[1]
    br = 1000
    grid = n // br

    def body(x_ref, w_ref, o_ref):
        o_ref[...] = lax.dot_general(
            x_ref[...], w_ref[...], (((1,), (0,)), ((), ())),
            preferred_element_type=jnp.float32,
            precision=lax.Precision.HIGHEST)

    return pl.pallas_call(
        body,
        grid=(grid,),
        in_specs=[pl.BlockSpec((br, f), lambda i: (i, 0)),
                  pl.BlockSpec((f, h), lambda i: (0, 0))],
        out_specs=pl.BlockSpec((br, h), lambda i: (i, 0)),
        out_shape=jax.ShapeDtypeStruct((n, h), jnp.float32),
    )(x, W)


def _bcast_lane(v, j):
    idx = jnp.full((_L, 1), j, jnp.int32)
    return lax.gather(
        v, idx,
        lax.GatherDimensionNumbers(offset_dims=(), collapsed_slice_dims=(0,),
                                   start_index_map=(0,)),
        (1,), mode=lax.GatherScatterMode.PROMISE_IN_BOUNDS)


def _edge_pass_sc(h, pk, n_nodes, ep):
    per_w = ep // _NW
    n_chunks = per_w // _CHUNK
    rows_per_tile = -(-n_nodes // (_NS * 8)) * 8
    n_pad = rows_per_tile * _NS
    hdim = h.shape[1]

    mesh = plsc.VectorSubcoreMesh(core_axis_name="c", subcore_axis_name="s",
                                  num_cores=_NC, num_subcores=_NS)
    cp = pltpu.CompilerParams()
    if "needs_layout_passes" in pltpu.CompilerParams.__dataclass_fields__:
        cp = dataclasses.replace(cp, needs_layout_passes=False)
    if "use_tc_tiling_on_sc" in pltpu.CompilerParams.__dataclass_fields__:
        cp = dataclasses.replace(cp, use_tc_tiling_on_sc=False)

    @functools.partial(
        pl.kernel,
        compiler_params=cp,
        out_type=jax.ShapeDtypeStruct((_NC, n_pad, hdim), jnp.float32),
        mesh=mesh,
        scratch_types=[
            pltpu.VMEM((3, _CHUNK), jnp.int32),
            pltpu.VMEM((3, _CHUNK), jnp.int32),
            pltpu.VMEM((3, _CHUNK), jnp.int32),
            pltpu.VMEM((_CHUNK,), jnp.int32),
            pltpu.VMEM((_CHUNK,), jnp.int32),
            pltpu.VMEM((_CHUNK,), jnp.int32),
            pltpu.VMEM((_CHUNK, hdim), jnp.float32),
            pltpu.VMEM((_CHUNK, hdim), jnp.float32),
            pltpu.VMEM((_CHUNK, hdim), jnp.float32),
            pltpu.VMEM_SHARED((n_pad, hdim), jnp.float32),
            pltpu.SemaphoreType.DMA,
            pltpu.SemaphoreType.DMA,
            pltpu.SemaphoreType.DMA,
            pltpu.SemaphoreType.DMA,
            pltpu.SemaphoreType.DMA,
            pltpu.SemaphoreType.DMA,
        ],
    )
    def ker(h_hbm, pk_hbm, out_hbm,
            pk0, pk1, pk2, d0, d1, d2, r0, r1, r2, agg_sh,
            is0, is1, is2, gs0, gs1, gs2):
        pks, ds, rs = [pk0, pk1, pk2], [d0, d1, d2], [r0, r1, r2]
        iss, gss = [is0, is1, is2], [gs0, gs1, gs2]
        c = lax.axis_index("c")
        s = lax.axis_index("s")
        wid = c * _NS + s
        base = wid * n_chunks

        def idx_start(b, ci):
            pltpu.make_async_copy(pk_hbm.at[base + ci], pks[b],
                                  iss[b]).start()

        def idx_wait(b, ci):
            pltpu.make_async_copy(pk_hbm.at[base + ci], pks[b],
                                  iss[b]).wait()

        def gather_start(b):
            pltpu.make_async_copy(h_hbm.at[pks[b].at[0]], rs[b],
                                  gss[b]).start()

        def gather_wait(b):
            pltpu.make_async_copy(h_hbm.at[pks[b].at[0]], rs[b],
                                  gss[b]).wait()

        @pl.loop(0, _CHUNK)
        def _(i):
            for j in range(hdim // _L):
                r0[i, pl.ds(j * _L, _L)] = jnp.zeros((_L,), jnp.float32)

        @pl.loop(0, rows_per_tile // _CHUNK)
        def _(i):
            pltpu.sync_copy(
                r0, agg_sh.at[pl.ds(s * rows_per_tile + i * _CHUNK, _CHUNK)])
        plsc.subcore_barrier()

        idx_start(0, 0)
        idx_start(1, 1)
        idx_wait(0, 0)
        gather_start(0)

        @pl.loop(0, n_chunks // 3)
        def _(t):
            for k in range(3):
                b, b1, b2 = k, (k + 1) % 3, (k + 2) % 3
                ci = t * 3 + k
                gather_wait(b)

                @pl.when(ci + 1 < n_chunks)
                def _():
                    idx_wait(b1, ci + 1)
                    gather_start(b1)

                @pl.when(ci + 2 < n_chunks)
                def _():
                    idx_start(b2, ci + 2)

                rows_v, pk_v = rs[b], pks[b]
                for g in range(_CHUNK // _L):
                    ds[b][pl.ds(g * _L, _L)] = pk_v[1, pl.ds(g * _L, _L)]

                for g in range(_CHUNK // _L):
                    w16 = plsc.bitcast(pk_v[2, pl.ds(g * _L, _L)],
                                       jnp.float32)
                    for j in range(_L):
                        e = g * _L + j
                        wvec = _bcast_lane(w16, j)
                        for half in range(hdim // _L):
                            rows_v[e, pl.ds(half * _L, _L)] = (
                                rows_v[e, pl.ds(half * _L, _L)] * wvec)

                pltpu.sync_copy(rows_v, agg_sh.at[ds[b]], add=True)

        plsc.subcore_barrier()
        pltpu.sync_copy(
            agg_sh.at[pl.ds(s * rows_per_tile, rows_per_tile)],
            out_hbm.at[c].at[pl.ds(s * rows_per_tile, rows_per_tile)])

    return ker(h, pk)


def _elu(t):
    return jnp.where(t > 0, t, jnp.exp(jnp.minimum(t, 0.0)) - 1.0)


def _mid_layer_tc(parts, b1, W2, n):
    hdim = parts.shape[2]
    h2 = W2.shape[1]
    br = 1000
    grid = n // br

    def body(p_ref, b_ref, w_ref, o_ref):
        t = p_ref[0] + p_ref[1] + b_ref[...]
        t = _elu(t)
        o_ref[...] = lax.dot_general(
            t, w_ref[...], (((1,), (0,)), ((), ())),
            preferred_element_type=jnp.float32,
            precision=lax.Precision.HIGHEST)

    return pl.pallas_call(
        body,
        grid=(grid,),
        in_specs=[pl.BlockSpec((2, br, hdim), lambda i: (0, i, 0)),
                  pl.BlockSpec((1, hdim), lambda i: (0, 0)),
                  pl.BlockSpec((hdim, h2), lambda i: (0, 0))],
        out_specs=pl.BlockSpec((br, h2), lambda i: (i, 0)),
        out_shape=jax.ShapeDtypeStruct((n, h2), jnp.float32),
    )(parts, b1.reshape(1, -1), W2)


def _head_tc(parts, b2, fc1_W, fc1_b, fc2_W, fc2_b, n):
    hdim = parts.shape[2]
    fc1 = fc1_W.shape[1]
    out_dim = fc2_W.shape[1]
    br = 1000
    grid = n // br

    def body(p_ref, b_ref, w1_ref, c1_ref, w2_ref, c2_ref, o_ref, acc_ref):
        i = pl.program_id(0)
        t = p_ref[0] + p_ref[1] + b_ref[...]
        t = _elu(t)
        part = jnp.sum(t, axis=0, keepdims=True)

        @pl.when(i == 0)
        def _():
            acc_ref[...] = part

        @pl.when(i > 0)
        def _():
            acc_ref[...] = acc_ref[...] + part

        @pl.when(i == pl.num_programs(0) - 1)
        def _():
            z = lax.dot_general(
                acc_ref[...], w1_ref[...], (((1,), (0,)), ((), ())),
                preferred_element_type=jnp.float32,
                precision=lax.Precision.HIGHEST) + c1_ref[...]
            z = jnp.maximum(z, 0.0)
            y = lax.dot_general(
                z, w2_ref[...], (((1,), (0,)), ((), ())),
                preferred_element_type=jnp.float32,
                precision=lax.Precision.HIGHEST) + c2_ref[...]
            o_ref[...] = 1.0 / (1.0 + jnp.exp(-y))

    return pl.pallas_call(
        body,
        grid=(grid,),
        in_specs=[pl.BlockSpec((2, br, hdim), lambda i: (0, i, 0)),
                  pl.BlockSpec((1, hdim), lambda i: (0, 0)),
                  pl.BlockSpec((hdim, fc1), lambda i: (0, 0)),
                  pl.BlockSpec((1, fc1), lambda i: (0, 0)),
                  pl.BlockSpec((fc1, out_dim), lambda i: (0, 0)),
                  pl.BlockSpec((1, out_dim), lambda i: (0, 0))],
        out_specs=pl.BlockSpec((1, out_dim), lambda i: (0, 0)),
        out_shape=jax.ShapeDtypeStruct((1, out_dim), jnp.float32),
        scratch_shapes=[pltpu.VMEM((1, hdim), jnp.float32)],
    )(parts, b2.reshape(1, -1), fc1_W, fc1_b.reshape(1, -1),
      fc2_W, fc2_b.reshape(1, -1))


def kernel(x, edge_index, edge_weight, W1, b1, W2, b2,
           fc1_W, fc1_b, fc2_W, fc2_b):
    n = x.shape[0]
    e = edge_index.shape[1]
    src = edge_index[0]
    dst = edge_index[1]

    unit = _NW * _CHUNK * 3
    ep = ((e + unit - 1) // unit) * unit
    pad = ep - e
    if pad:
        src = jnp.concatenate([src, jnp.zeros((pad,), jnp.int32)])
        dst = jnp.concatenate([dst, jnp.zeros((pad,), jnp.int32)])
        edge_weight = jnp.concatenate(
            [edge_weight, jnp.zeros((pad,), jnp.float32)])

    pk = jnp.stack(
        [src.reshape(-1, _CHUNK), dst.reshape(-1, _CHUNK),
         lax.bitcast_convert_type(edge_weight, jnp.int32).reshape(-1, _CHUNK)],
        axis=1)

    h1 = _matmul_tc(x, W1)
    p1 = _edge_pass_sc(h1, pk, n, ep)
    h2 = _mid_layer_tc(p1, b1, W2, n)
    p2 = _edge_pass_sc(h2, pk, n, ep)
    out = _head_tc(p2, b2, fc1_W, fc1_b, fc2_W, fc2_b, n)
    return out.reshape(-1)


# speedup vs baseline: 7.1901x; 1.0010x over previous
import dataclasses
import functools

import jax
import jax.numpy as jnp
from jax import lax
from jax.experimental import pallas as pl
from jax.experimental.pallas import tpu as pltpu
from jax.experimental.pallas import tpu_sc as plsc

_NC = 2
_NS = 16
_L = 16
_NW = _NC * _NS
_CHUNK = 128


def _matmul_tc(x, W):
    n, f = x.shape
    h = W.shape[1]
    br = 1000
    grid = n // br

    def body(x_ref, w_ref, o_ref):
        o_ref[...] = lax.dot_general(
            x_ref[...], w_ref[...], (((1,), (0,)), ((), ())),
            preferred_element_type=jnp.float32,
            precision=lax.Precision.HIGHEST)

    return pl.pallas_call(
        body,
        grid=(grid,),
        in_specs=[pl.BlockSpec((br, f), lambda i: (i, 0)),
                  pl.BlockSpec((f, h), lambda i: (0, 0))],
        out_specs=pl.BlockSpec((br, h), lambda i: (i, 0)),
        out_shape=jax.ShapeDtypeStruct((n, h), jnp.float32),
    )(x, W)


def _bcast_lane(v, j):
    idx = jnp.full((_L, 1), j, jnp.int32)
    return lax.gather(
        v, idx,
        lax.GatherDimensionNumbers(offset_dims=(), collapsed_slice_dims=(0,),
                                   start_index_map=(0,)),
        (1,), mode=lax.GatherScatterMode.PROMISE_IN_BOUNDS)


def _edge_pass_sc(h, pk, n_nodes, ep):
    per_w = ep // _NW
    n_chunks = per_w // _CHUNK
    rows_per_tile = -(-n_nodes // (_NS * 8)) * 8
    n_pad = rows_per_tile * _NS
    hdim = h.shape[1]

    mesh = plsc.VectorSubcoreMesh(core_axis_name="c", subcore_axis_name="s",
                                  num_cores=_NC, num_subcores=_NS)
    cp = pltpu.CompilerParams()
    if "needs_layout_passes" in pltpu.CompilerParams.__dataclass_fields__:
        cp = dataclasses.replace(cp, needs_layout_passes=False)
    if "use_tc_tiling_on_sc" in pltpu.CompilerParams.__dataclass_fields__:
        cp = dataclasses.replace(cp, use_tc_tiling_on_sc=False)

    @functools.partial(
        pl.kernel,
        compiler_params=cp,
        out_type=jax.ShapeDtypeStruct((_NC, n_pad, hdim), jnp.float32),
        mesh=mesh,
        scratch_types=[
            pltpu.VMEM((3, _CHUNK), jnp.int32),
            pltpu.VMEM((3, _CHUNK), jnp.int32),
            pltpu.VMEM((3, _CHUNK), jnp.int32),
            pltpu.VMEM((_CHUNK,), jnp.int32),
            pltpu.VMEM((_CHUNK,), jnp.int32),
            pltpu.VMEM((_CHUNK,), jnp.int32),
            pltpu.VMEM((_CHUNK, hdim), jnp.float32),
            pltpu.VMEM((_CHUNK, hdim), jnp.float32),
            pltpu.VMEM((_CHUNK, hdim), jnp.float32),
            pltpu.VMEM_SHARED((n_pad, hdim), jnp.float32),
            pltpu.SemaphoreType.DMA,
            pltpu.SemaphoreType.DMA,
            pltpu.SemaphoreType.DMA,
            pltpu.SemaphoreType.DMA,
            pltpu.SemaphoreType.DMA,
            pltpu.SemaphoreType.DMA,
            pltpu.SemaphoreType.DMA,
            pltpu.SemaphoreType.DMA,
            pltpu.SemaphoreType.DMA,
        ],
    )
    def ker(h_hbm, pk_hbm, out_hbm,
            pk0, pk1, pk2, d0, d1, d2, r0, r1, r2, agg_sh,
            is0, is1, is2, gs0, gs1, gs2, ss0, ss1, ss2):
        pks, ds, rs = [pk0, pk1, pk2], [d0, d1, d2], [r0, r1, r2]
        iss, gss, sss = [is0, is1, is2], [gs0, gs1, gs2], [ss0, ss1, ss2]
        c = lax.axis_index("c")
        s = lax.axis_index("s")
        wid = c * _NS + s
        base = wid * n_chunks

        def idx_start(b, ci):
            pltpu.make_async_copy(pk_hbm.at[base + ci], pks[b],
                                  iss[b]).start()

        def idx_wait(b, ci):
            pltpu.make_async_copy(pk_hbm.at[base + ci], pks[b],
                                  iss[b]).wait()

        def gather_start(b):
            pltpu.make_async_copy(h_hbm.at[pks[b].at[0]], rs[b],
                                  gss[b]).start()

        def gather_wait(b):
            pltpu.make_async_copy(h_hbm.at[pks[b].at[0]], rs[b],
                                  gss[b]).wait()

        def scatter_start(b):
            pltpu.async_copy(rs[b], agg_sh.at[ds[b]], sss[b], add=True)

        def scatter_wait(b):
            pltpu.make_async_copy(rs[b], agg_sh.at[ds[b]], sss[b]).wait()

        @pl.loop(0, _CHUNK)
        def _(i):
            for j in range(hdim // _L):
                r0[i, pl.ds(j * _L, _L)] = jnp.zeros((_L,), jnp.float32)

        @pl.loop(0, rows_per_tile // _CHUNK)
        def _(i):
            pltpu.sync_copy(
                r0, agg_sh.at[pl.ds(s * rows_per_tile + i * _CHUNK, _CHUNK)])
        plsc.subcore_barrier()

        idx_start(0, 0)
        idx_start(1, 1)
        idx_wait(0, 0)
        gather_start(0)

        @pl.loop(0, n_chunks // 3)
        def _(t):
            for k in range(3):
                b, b1, b2 = k, (k + 1) % 3, (k + 2) % 3
                ci = t * 3 + k
                gather_wait(b)

                @pl.when(ci + 1 < n_chunks)
                def _():
                    idx_wait(b1, ci + 1)
                    gather_start(b1)

                rows_v, pk_v = rs[b], pks[b]
                for g in range(_CHUNK // _L):
                    ds[b][pl.ds(g * _L, _L)] = pk_v[1, pl.ds(g * _L, _L)]

                for g in range(_CHUNK // _L):
                    w16 = plsc.bitcast(pk_v[2, pl.ds(g * _L, _L)],
                                       jnp.float32)
                    for j in range(_L):
                        e = g * _L + j
                        wvec = _bcast_lane(w16, j)
                        for half in range(hdim // _L):
                            rows_v[e, pl.ds(half * _L, _L)] = (
                                rows_v[e, pl.ds(half * _L, _L)] * wvec)

                @pl.when(ci >= 1)
                def _():
                    scatter_wait(b2)

                @pl.when(ci + 2 < n_chunks)
                def _():
                    idx_start(b2, ci + 2)

                scatter_start(b)

        scatter_wait((n_chunks - 1) % 3)
        plsc.subcore_barrier()
        pltpu.sync_copy(
            agg_sh.at[pl.ds(s * rows_per_tile, rows_per_tile)],
            out_hbm.at[c].at[pl.ds(s * rows_per_tile, rows_per_tile)])

    return ker(h, pk)


def _elu(t):
    return jnp.where(t > 0, t, jnp.exp(jnp.minimum(t, 0.0)) - 1.0)


def _mid_layer_tc(parts, b1, W2, n):
    hdim = parts.shape[2]
    h2 = W2.shape[1]
    br = 1000
    grid = n // br

    def body(p_ref, b_ref, w_ref, o_ref):
        t = p_ref[0] + p_ref[1] + b_ref[...]
        t = _elu(t)
        o_ref[...] = lax.dot_general(
            t, w_ref[...], (((1,), (0,)), ((), ())),
            preferred_element_type=jnp.float32,
            precision=lax.Precision.HIGHEST)

    return pl.pallas_call(
        body,
        grid=(grid,),
        in_specs=[pl.BlockSpec((2, br, hdim), lambda i: (0, i, 0)),
                  pl.BlockSpec((1, hdim), lambda i: (0, 0)),
                  pl.BlockSpec((hdim, h2), lambda i: (0, 0))],
        out_specs=pl.BlockSpec((br, h2), lambda i: (i, 0)),
        out_shape=jax.ShapeDtypeStruct((n, h2), jnp.float32),
    )(parts, b1.reshape(1, -1), W2)


def _head_tc(parts, b2, fc1_W, fc1_b, fc2_W, fc2_b, n):
    hdim = parts.shape[2]
    fc1 = fc1_W.shape[1]
    out_dim = fc2_W.shape[1]
    br = 1000
    grid = n // br

    def body(p_ref, b_ref, w1_ref, c1_ref, w2_ref, c2_ref, o_ref, acc_ref):
        i = pl.program_id(0)
        t = p_ref[0] + p_ref[1] + b_ref[...]
        t = _elu(t)
        part = jnp.sum(t, axis=0, keepdims=True)

        @pl.when(i == 0)
        def _():
            acc_ref[...] = part

        @pl.when(i > 0)
        def _():
            acc_ref[...] = acc_ref[...] + part

        @pl.when(i == pl.num_programs(0) - 1)
        def _():
            z = lax.dot_general(
                acc_ref[...], w1_ref[...], (((1,), (0,)), ((), ())),
                preferred_element_type=jnp.float32,
                precision=lax.Precision.HIGHEST) + c1_ref[...]
            z = jnp.maximum(z, 0.0)
            y = lax.dot_general(
                z, w2_ref[...], (((1,), (0,)), ((), ())),
                preferred_element_type=jnp.float32,
                precision=lax.Precision.HIGHEST) + c2_ref[...]
            o_ref[...] = 1.0 / (1.0 + jnp.exp(-y))

    return pl.pallas_call(
        body,
        grid=(grid,),
        in_specs=[pl.BlockSpec((2, br, hdim), lambda i: (0, i, 0)),
                  pl.BlockSpec((1, hdim), lambda i: (0, 0)),
                  pl.BlockSpec((hdim, fc1), lambda i: (0, 0)),
                  pl.BlockSpec((1, fc1), lambda i: (0, 0)),
                  pl.BlockSpec((fc1, out_dim), lambda i: (0, 0)),
                  pl.BlockSpec((1, out_dim), lambda i: (0, 0))],
        out_specs=pl.BlockSpec((1, out_dim), lambda i: (0, 0)),
        out_shape=jax.ShapeDtypeStruct((1, out_dim), jnp.float32),
        scratch_shapes=[pltpu.VMEM((1, hdim), jnp.float32)],
    )(parts, b2.reshape(1, -1), fc1_W, fc1_b.reshape(1, -1),
      fc2_W, fc2_b.reshape(1, -1))


def kernel(x, edge_index, edge_weight, W1, b1, W2, b2,
           fc1_W, fc1_b, fc2_W, fc2_b):
    n = x.shape[0]
    e = edge_index.shape[1]
    src = edge_index[0]
    dst = edge_index[1]

    unit = _NW * _CHUNK * 3
    ep = ((e + unit - 1) // unit) * unit
    pad = ep - e
    if pad:
        src = jnp.concatenate([src, jnp.zeros((pad,), jnp.int32)])
        dst = jnp.concatenate([dst, jnp.zeros((pad,), jnp.int32)])
        edge_weight = jnp.concatenate(
            [edge_weight, jnp.zeros((pad,), jnp.float32)])

    pk = jnp.stack(
        [src.reshape(-1, _CHUNK), dst.reshape(-1, _CHUNK),
         lax.bitcast_convert_type(edge_weight, jnp.int32).reshape(-1, _CHUNK)],
        axis=1)

    h1 = _matmul_tc(x, W1)
    p1 = _edge_pass_sc(h1, pk, n, ep)
    h2 = _mid_layer_tc(p1, b1, W2, n)
    p2 = _edge_pass_sc(h2, pk, n, ep)
    out = _head_tc(p2, b2, fc1_W, fc1_b, fc2_W, fc2_b, n)
    return out.reshape(-1)


# speedup vs baseline: 8.1499x; 1.1335x over previous
import dataclasses
import functools

import jax
import jax.numpy as jnp
from jax import lax
from jax.experimental import pallas as pl
from jax.experimental.pallas import tpu as pltpu
from jax.experimental.pallas import tpu_sc as plsc

_NC = 2
_NS = 16
_L = 16
_NW = _NC * _NS
_CHUNK = 128


def _matmul_tc(x, W):
    n, f = x.shape
    h = W.shape[1]
    br = 1000
    grid = n // br

    def body(x_ref, w_ref, o_ref):
        o_ref[...] = lax.dot_general(
            x_ref[...], w_ref[...], (((1,), (0,)), ((), ())),
            preferred_element_type=jnp.float32,
            precision=lax.Precision.HIGHEST)

    return pl.pallas_call(
        body,
        grid=(grid,),
        in_specs=[pl.BlockSpec((br, f), lambda i: (i, 0)),
                  pl.BlockSpec((f, h), lambda i: (0, 0))],
        out_specs=pl.BlockSpec((br, h), lambda i: (i, 0)),
        out_shape=jax.ShapeDtypeStruct((n, h), jnp.float32),
    )(x, W)


def _bcast_lane(v, j):
    idx = jnp.full((_L, 1), j, jnp.int32)
    return lax.gather(
        v, idx,
        lax.GatherDimensionNumbers(offset_dims=(), collapsed_slice_dims=(0,),
                                   start_index_map=(0,)),
        (1,), mode=lax.GatherScatterMode.PROMISE_IN_BOUNDS)


def _edge_pass_sc(h, pk, n_nodes, ep):
    per_w = ep // _NW
    n_chunks = per_w // _CHUNK
    rows_per_tile = -(-n_nodes // (_NS * 8)) * 8
    n_pad = rows_per_tile * _NS
    hdim = h.shape[1]

    mesh = plsc.VectorSubcoreMesh(core_axis_name="c", subcore_axis_name="s",
                                  num_cores=_NC, num_subcores=_NS)
    cp = pltpu.CompilerParams()
    if "needs_layout_passes" in pltpu.CompilerParams.__dataclass_fields__:
        cp = dataclasses.replace(cp, needs_layout_passes=False)
    if "use_tc_tiling_on_sc" in pltpu.CompilerParams.__dataclass_fields__:
        cp = dataclasses.replace(cp, use_tc_tiling_on_sc=False)

    @functools.partial(
        pl.kernel,
        compiler_params=cp,
        out_type=jax.ShapeDtypeStruct((_NC, n_pad, hdim), jnp.float32),
        mesh=mesh,
        scratch_types=(
            [pltpu.VMEM((3, _CHUNK), jnp.int32)] * 4 +
            [pltpu.VMEM((_CHUNK,), jnp.int32)] * 4 +
            [pltpu.VMEM((_CHUNK, hdim), jnp.float32)] * 4 +
            [pltpu.VMEM_SHARED((n_pad, hdim), jnp.float32)] +
            [pltpu.SemaphoreType.DMA] * 12
        ),
    )
    def ker(h_hbm, pk_hbm, out_hbm,
            pk0, pk1, pk2, pk3, d0, d1, d2, d3, r0, r1, r2, r3, agg_sh,
            is0, is1, is2, is3, gs0, gs1, gs2, gs3, ss0, ss1, ss2, ss3):
        pks, ds = [pk0, pk1, pk2, pk3], [d0, d1, d2, d3]
        rs = [r0, r1, r2, r3]
        iss, gss = [is0, is1, is2, is3], [gs0, gs1, gs2, gs3]
        sss = [ss0, ss1, ss2, ss3]
        c = lax.axis_index("c")
        s = lax.axis_index("s")
        wid = c * _NS + s
        base = wid * n_chunks

        def idx_start(b, ci):
            pltpu.make_async_copy(pk_hbm.at[base + ci], pks[b],
                                  iss[b]).start()

        def idx_wait(b, ci):
            pltpu.make_async_copy(pk_hbm.at[base + ci], pks[b],
                                  iss[b]).wait()

        def gather_start(b):
            pltpu.make_async_copy(h_hbm.at[pks[b].at[0]], rs[b],
                                  gss[b]).start()

        def gather_wait(b):
            pltpu.make_async_copy(h_hbm.at[pks[b].at[0]], rs[b],
                                  gss[b]).wait()

        def scatter_start(b):
            pltpu.async_copy(rs[b], agg_sh.at[ds[b]], sss[b], add=True)

        def scatter_wait(b):
            pltpu.make_async_copy(rs[b], agg_sh.at[ds[b]], sss[b]).wait()

        @pl.loop(0, _CHUNK)
        def _(i):
            for j in range(hdim // _L):
                r0[i, pl.ds(j * _L, _L)] = jnp.zeros((_L,), jnp.float32)

        @pl.loop(0, rows_per_tile // _CHUNK)
        def _(i):
            pltpu.sync_copy(
                r0, agg_sh.at[pl.ds(s * rows_per_tile + i * _CHUNK, _CHUNK)])
        plsc.subcore_barrier()

        idx_start(0, 0)
        idx_start(1, 1)
        idx_start(2, 2)
        idx_wait(0, 0)
        gather_start(0)

        @pl.loop(0, n_chunks // 4)
        def _(t):
            for k in range(4):
                b, b1, b3 = k, (k + 1) % 4, (k + 3) % 4
                ci = t * 4 + k
                gather_wait(b)

                @pl.when(ci + 1 < n_chunks)
                def _():
                    idx_wait(b1, ci + 1)
                    gather_start(b1)

                rows_v, pk_v = rs[b], pks[b]
                for g in range(_CHUNK // _L):
                    ds[b][pl.ds(g * _L, _L)] = pk_v[1, pl.ds(g * _L, _L)]

                for g in range(_CHUNK // _L):
                    w16 = plsc.bitcast(pk_v[2, pl.ds(g * _L, _L)],
                                       jnp.float32)
                    for j in range(_L):
                        e = g * _L + j
                        wvec = _bcast_lane(w16, j)
                        for half in range(hdim // _L):
                            rows_v[e, pl.ds(half * _L, _L)] = (
                                rows_v[e, pl.ds(half * _L, _L)] * wvec)

                @pl.when(ci >= 1)
                def _():
                    scatter_wait(b3)

                @pl.when(ci + 3 < n_chunks)
                def _():
                    idx_start(b3, ci + 3)

                scatter_start(b)

        scatter_wait((n_chunks - 1) % 4)
        plsc.subcore_barrier()
        pltpu.sync_copy(
            agg_sh.at[pl.ds(s * rows_per_tile, rows_per_tile)],
            out_hbm.at[c].at[pl.ds(s * rows_per_tile, rows_per_tile)])

    return ker(h, pk)


def _elu(t):
    return jnp.where(t > 0, t, jnp.exp(jnp.minimum(t, 0.0)) - 1.0)


def _mid_layer_tc(parts, b1, W2, n):
    hdim = parts.shape[2]
    h2 = W2.shape[1]
    br = 1000
    grid = n // br

    def body(p_ref, b_ref, w_ref, o_ref):
        t = p_ref[0] + p_ref[1] + b_ref[...]
        t = _elu(t)
        o_ref[...] = lax.dot_general(
            t, w_ref[...], (((1,), (0,)), ((), ())),
            preferred_element_type=jnp.float32,
            precision=lax.Precision.HIGHEST)

    return pl.pallas_call(
        body,
        grid=(grid,),
        in_specs=[pl.BlockSpec((2, br, hdim), lambda i: (0, i, 0)),
                  pl.BlockSpec((1, hdim), lambda i: (0, 0)),
                  pl.BlockSpec((hdim, h2), lambda i: (0, 0))],
        out_specs=pl.BlockSpec((br, h2), lambda i: (i, 0)),
        out_shape=jax.ShapeDtypeStruct((n, h2), jnp.float32),
    )(parts, b1.reshape(1, -1), W2)


def _head_tc(parts, b2, fc1_W, fc1_b, fc2_W, fc2_b, n):
    hdim = parts.shape[2]
    fc1 = fc1_W.shape[1]
    out_dim = fc2_W.shape[1]
    br = 1000
    grid = n // br

    def body(p_ref, b_ref, w1_ref, c1_ref, w2_ref, c2_ref, o_ref, acc_ref):
        i = pl.program_id(0)
        t = p_ref[0] + p_ref[1] + b_ref[...]
        t = _elu(t)
        part = jnp.sum(t, axis=0, keepdims=True)

        @pl.when(i == 0)
        def _():
            acc_ref[...] = part

        @pl.when(i > 0)
        def _():
            acc_ref[...] = acc_ref[...] + part

        @pl.when(i == pl.num_programs(0) - 1)
        def _():
            z = lax.dot_general(
                acc_ref[...], w1_ref[...], (((1,), (0,)), ((), ())),
                preferred_element_type=jnp.float32,
                precision=lax.Precision.HIGHEST) + c1_ref[...]
            z = jnp.maximum(z, 0.0)
            y = lax.dot_general(
                z, w2_ref[...], (((1,), (0,)), ((), ())),
                preferred_element_type=jnp.float32,
                precision=lax.Precision.HIGHEST) + c2_ref[...]
            o_ref[...] = 1.0 / (1.0 + jnp.exp(-y))

    return pl.pallas_call(
        body,
        grid=(grid,),
        in_specs=[pl.BlockSpec((2, br, hdim), lambda i: (0, i, 0)),
                  pl.BlockSpec((1, hdim), lambda i: (0, 0)),
                  pl.BlockSpec((hdim, fc1), lambda i: (0, 0)),
                  pl.BlockSpec((1, fc1), lambda i: (0, 0)),
                  pl.BlockSpec((fc1, out_dim), lambda i: (0, 0)),
                  pl.BlockSpec((1, out_dim), lambda i: (0, 0))],
        out_specs=pl.BlockSpec((1, out_dim), lambda i: (0, 0)),
        out_shape=jax.ShapeDtypeStruct((1, out_dim), jnp.float32),
        scratch_shapes=[pltpu.VMEM((1, hdim), jnp.float32)],
    )(parts, b2.reshape(1, -1), fc1_W, fc1_b.reshape(1, -1),
      fc2_W, fc2_b.reshape(1, -1))


def kernel(x, edge_index, edge_weight, W1, b1, W2, b2,
           fc1_W, fc1_b, fc2_W, fc2_b):
    n = x.shape[0]
    e = edge_index.shape[1]
    src = edge_index[0]
    dst = edge_index[1]

    unit = _NW * _CHUNK * 4
    ep = ((e + unit - 1) // unit) * unit
    pad = ep - e
    if pad:
        src = jnp.concatenate([src, jnp.zeros((pad,), jnp.int32)])
        dst = jnp.concatenate([dst, jnp.zeros((pad,), jnp.int32)])
        edge_weight = jnp.concatenate(
            [edge_weight, jnp.zeros((pad,), jnp.float32)])

    pk = jnp.stack(
        [src.reshape(-1, _CHUNK), dst.reshape(-1, _CHUNK),
         lax.bitcast_convert_type(edge_weight, jnp.int32).reshape(-1, _CHUNK)],
        axis=1)

    h1 = _matmul_tc(x, W1)
    p1 = _edge_pass_sc(h1, pk, n, ep)
    h2 = _mid_layer_tc(p1, b1, W2, n)
    p2 = _edge_pass_sc(h2, pk, n, ep)
    out = _head_tc(p2, b2, fc1_W, fc1_b, fc2_W, fc2_b, n)
    return out.reshape(-1)
